# Initial kernel scaffold; baseline (speedup 1.0000x reference)
import jax
import jax.numpy as jnp
from jax.experimental import pallas as pl


def kernel(feature, edge_index, W1, b1, Wm, bm, Wl, bl, Wd1, bd1, Wd2, bd2):
    raise NotImplementedError("write your pallas kernel here")


# speedup vs baseline: 14.9255x; 14.9255x over previous
import functools

import jax
import jax.numpy as jnp
from jax import lax
from jax.experimental import pallas as pl
from jax.experimental.pallas import tpu as pltpu
from jax.experimental.pallas import tpu_sc as plsc

N = 10000
E = 320000
CHUNK = 128
NCHUNKS = E // CHUNK
NW = 32
ROWS_LO = 624
ROWS_HI = 640
ROW_SPLIT = 15 * ROWS_LO
D1_LO = 640
NPAD1 = 16 * D1_LO

_MESH = plsc.VectorSubcoreMesh(core_axis_name="c", subcore_axis_name="s")


@functools.partial(
    pl.kernel,
    mesh=_MESH,
    out_type=jax.ShapeDtypeStruct((2, NPAD1), jnp.float32),
    scratch_types=[
        pltpu.VMEM((CHUNK,), jnp.int32),
        pltpu.VMEM((CHUNK,), jnp.float32),
        pltpu.VMEM((D1_LO,), jnp.float32),
        pltpu.VMEM_SHARED((NPAD1,), jnp.float32),
    ],
)
def _sc_degree(dst_hbm, ones_hbm, zcol_hbm, deg_out, idx_v, ones_v, z_v, acc_sh):
    cid = lax.axis_index("c")
    sid = lax.axis_index("s")
    wid = sid * 2 + cid

    pltpu.sync_copy(ones_hbm, ones_v)
    pltpu.sync_copy(zcol_hbm, z_v)
    pltpu.sync_copy(z_v, acc_sh.at[pl.ds(sid * D1_LO, D1_LO)])
    plsc.subcore_barrier()

    n_i = (NCHUNKS - wid + NW - 1) // NW

    def body(i, _):
        base = (wid + i * NW) * CHUNK
        pltpu.sync_copy(dst_hbm.at[pl.ds(base, CHUNK)], idx_v)
        pltpu.sync_copy(ones_v, acc_sh.at[idx_v], add=True)
        return 0

    lax.fori_loop(0, n_i, body, 0)
    plsc.subcore_barrier()

    pltpu.sync_copy(acc_sh.at[pl.ds(sid * D1_LO, D1_LO)],
                    deg_out.at[cid].at[pl.ds(sid * D1_LO, D1_LO)])


@functools.partial(
    pl.kernel,
    mesh=_MESH,
    out_type=jax.ShapeDtypeStruct((2, N, 128), jnp.float32),
    scratch_types=[
        pltpu.VMEM((CHUNK,), jnp.int32),
        pltpu.VMEM((CHUNK,), jnp.int32),
        pltpu.VMEM((CHUNK, 128), jnp.float32),
        pltpu.VMEM_SHARED((N, 128), jnp.float32),
        pltpu.SemaphoreType.DMA,
    ],
)
def _sc_prop(g_hbm, src_hbm, dst_hbm, zrows_hbm, acc_out,
             src_v, dst_v, rows_v, acc_sh, sem):
    cid = lax.axis_index("c")
    sid = lax.axis_index("s")
    wid = sid * 2 + cid

    pltpu.sync_copy(zrows_hbm, rows_v)

    @pl.when(sid < 15)
    def _():
        row0 = sid * ROWS_LO
        for off, size in ((0, 128), (128, 128), (256, 128), (384, 128),
                          (512, 112)):
            pltpu.sync_copy(rows_v.at[pl.ds(0, size)],
                            acc_sh.at[pl.ds(row0 + off, size)])

    @pl.when(sid == 15)
    def _():
        for off in (0, 128, 256, 384, 512):
            pltpu.sync_copy(rows_v, acc_sh.at[pl.ds(ROW_SPLIT + off, 128)])

    plsc.subcore_barrier()

    n_i = (NCHUNKS - wid + NW - 1) // NW

    def body(i, _):
        base = (wid + i * NW) * CHUNK
        pltpu.sync_copy(src_hbm.at[pl.ds(base, CHUNK)], src_v)
        pltpu.sync_copy(dst_hbm.at[pl.ds(base, CHUNK)], dst_v)
        pltpu.async_copy(g_hbm.at[src_v], rows_v, sem).wait()
        pltpu.sync_copy(rows_v, acc_sh.at[dst_v], add=True)
        return 0

    lax.fori_loop(0, n_i, body, 0)
    plsc.subcore_barrier()

    @pl.when(sid < 15)
    def _():
        pltpu.sync_copy(acc_sh.at[pl.ds(sid * ROWS_LO, ROWS_LO)],
                        acc_out.at[cid].at[pl.ds(sid * ROWS_LO, ROWS_LO)])

    @pl.when(sid == 15)
    def _():
        pltpu.sync_copy(acc_sh.at[pl.ds(ROW_SPLIT, ROWS_HI)],
                        acc_out.at[cid].at[pl.ds(ROW_SPLIT, ROWS_HI)])


_BLK = 2000


def _row_spec(width):
    return pl.BlockSpec((_BLK, width), lambda i: (i, 0))


def _full_spec(shape):
    nd = len(shape)
    return pl.BlockSpec(shape, lambda i: (0,) * nd)


def _tc_call(body, out_shapes, in_specs, out_specs):
    return pl.pallas_call(
        body,
        grid=(N // _BLK,),
        in_specs=in_specs,
        out_specs=out_specs,
        out_shape=out_shapes,
    )


def _k1_body(degp_ref, x_ref, w_ref, dinv_ref, g_ref):
    deg = degp_ref[0] + degp_ref[1] + 1.0
    dinv = lax.rsqrt(deg)
    dinv_ref[...] = dinv
    g_ref[...] = dinv * jnp.dot(x_ref[...], w_ref[...],
                                preferred_element_type=jnp.float32)


def _k2_body(ap_ref, g_ref, dinv_ref, b1_ref, wml_ref, g2_ref):
    dinv = dinv_ref[...]
    h = jnp.tanh(dinv * (ap_ref[0] + ap_ref[1] + g_ref[...]) + b1_ref[...])
    g2_ref[...] = dinv * jnp.dot(h, wml_ref[...],
                                 preferred_element_type=jnp.float32)


def _k3_body(ap_ref, g_ref, dinv_ref, bm_ref, bl_ref, noise_ref, wd1_ref,
             mean_ref, logvar_ref, z_ref, g3_ref):
    dinv = dinv_ref[...]
    t = dinv * (ap_ref[0] + ap_ref[1] + g_ref[...])
    mean = t[:, :64] + bm_ref[...]
    logvar = t[:, 64:] + bl_ref[...]
    z = noise_ref[...] * jnp.exp(0.5 * logvar) + mean
    mean_ref[...] = mean
    logvar_ref[...] = logvar
    z_ref[...] = z
    g3_ref[...] = dinv * jnp.dot(z, wd1_ref[...],
                                 preferred_element_type=jnp.float32)


def _k4_body(ap_ref, g_ref, dinv_ref, bd1_ref, wd2_ref, g4_ref):
    dinv = dinv_ref[...]
    hd = jnp.tanh(dinv * (ap_ref[0] + ap_ref[1] + g_ref[...]) + bd1_ref[...])
    g4_ref[...] = dinv * jnp.dot(hd, wd2_ref[...],
                                 preferred_element_type=jnp.float32)


def _k5_body(ap_ref, g_ref, dinv_ref, bd2_ref, out_ref):
    out_ref[...] = (dinv_ref[...] * (ap_ref[0] + ap_ref[1] + g_ref[...])
                    + bd2_ref[...])


def _pair_spec(width):
    return pl.BlockSpec((2, _BLK, width), lambda i: (0, i, 0))


def kernel(feature, edge_index, W1, b1, Wm, bm, Wl, bl, Wd1, bd1, Wd2, bd2):
    f32 = jnp.float32
    src = edge_index[0]
    dst = edge_index[1]

    ones_col = jnp.ones((CHUNK,), f32)
    zcol = jnp.zeros((D1_LO,), f32)
    zrows = jnp.zeros((CHUNK, 128), f32)
    noise = jax.random.normal(jax.random.key(42), (N, 64), dtype=f32)
    Wml = jnp.concatenate([Wm, Wl], axis=1)

    deg_parts = _sc_degree(dst, ones_col, zcol)[:, :N, None]

    dinv, g1 = _tc_call(
        _k1_body,
        [jax.ShapeDtypeStruct((N, 1), f32), jax.ShapeDtypeStruct((N, 128), f32)],
        [_pair_spec(1), _row_spec(128), _full_spec((128, 128))],
        [_row_spec(1), _row_spec(128)],
    )(deg_parts, feature, W1)

    acc1 = _sc_prop(g1, src, dst, zrows)

    g2 = _tc_call(
        _k2_body,
        jax.ShapeDtypeStruct((N, 128), f32),
        [_pair_spec(128), _row_spec(128), _row_spec(1), _full_spec((128,)),
         _full_spec((128, 128))],
        _row_spec(128),
    )(acc1, g1, dinv, b1, Wml)

    acc2 = _sc_prop(g2, src, dst, zrows)

    mean, logvar, z, g3 = _tc_call(
        _k3_body,
        [jax.ShapeDtypeStruct((N, 64), f32), jax.ShapeDtypeStruct((N, 64), f32),
         jax.ShapeDtypeStruct((N, 64), f32), jax.ShapeDtypeStruct((N, 128), f32)],
        [_pair_spec(128), _row_spec(128), _row_spec(1), _full_spec((64,)),
         _full_spec((64,)), _row_spec(64), _full_spec((64, 128))],
        [_row_spec(64), _row_spec(64), _row_spec(64), _row_spec(128)],
    )(acc2, g2, dinv, bm, bl, noise, Wd1)

    acc3 = _sc_prop(g3, src, dst, zrows)

    g4 = _tc_call(
        _k4_body,
        jax.ShapeDtypeStruct((N, 128), f32),
        [_pair_spec(128), _row_spec(128), _row_spec(1), _full_spec((128,)),
         _full_spec((128, 128))],
        _row_spec(128),
    )(acc3, g3, dinv, bd1, Wd2)

    acc4 = _sc_prop(g4, src, dst, zrows)

    out = _tc_call(
        _k5_body,
        jax.ShapeDtypeStruct((N, 128), f32),
        [_pair_spec(128), _row_spec(128), _row_spec(1), _full_spec((128,))],
        _row_spec(128),
    )(acc4, g4, dinv, bd2)

    return (z, mean, logvar, out)


# speedup vs baseline: 24.9513x; 1.6717x over previous
import functools

import jax
import jax.numpy as jnp
from jax import lax
from jax.experimental import pallas as pl
from jax.experimental.pallas import tpu as pltpu
from jax.experimental.pallas import tpu_sc as plsc

N = 10000
E = 320000
CHUNK = 128
NCHUNKS = E // CHUNK
NW = 32
ROWS_LO = 624
ROWS_HI = 640
ROW_SPLIT = 15 * ROWS_LO
D1_LO = 640
NPAD1 = 16 * D1_LO

_MESH = plsc.VectorSubcoreMesh(core_axis_name="c", subcore_axis_name="s")


@functools.partial(
    pl.kernel,
    mesh=_MESH,
    out_type=jax.ShapeDtypeStruct((2, NPAD1), jnp.float32),
    scratch_types=[
        pltpu.VMEM((CHUNK,), jnp.int32),
        pltpu.VMEM((CHUNK,), jnp.float32),
        pltpu.VMEM((D1_LO,), jnp.float32),
        pltpu.VMEM_SHARED((NPAD1,), jnp.float32),
    ],
)
def _sc_degree(dst_hbm, ones_hbm, zcol_hbm, deg_out, idx_v, ones_v, z_v, acc_sh):
    cid = lax.axis_index("c")
    sid = lax.axis_index("s")
    wid = sid * 2 + cid

    pltpu.sync_copy(ones_hbm, ones_v)
    pltpu.sync_copy(zcol_hbm, z_v)
    pltpu.sync_copy(z_v, acc_sh.at[pl.ds(sid * D1_LO, D1_LO)])
    plsc.subcore_barrier()

    n_i = (NCHUNKS - wid + NW - 1) // NW

    def body(i, _):
        base = (wid + i * NW) * CHUNK
        pltpu.sync_copy(dst_hbm.at[pl.ds(base, CHUNK)], idx_v)
        pltpu.sync_copy(ones_v, acc_sh.at[idx_v], add=True)
        return 0

    lax.fori_loop(0, n_i, body, 0)
    plsc.subcore_barrier()

    pltpu.sync_copy(acc_sh.at[pl.ds(sid * D1_LO, D1_LO)],
                    deg_out.at[cid].at[pl.ds(sid * D1_LO, D1_LO)])


NFULL = (NCHUNKS // NW) * NW
NTAIL = NCHUNKS - NFULL
T_MAIN = NFULL // NW


@functools.partial(
    pl.kernel,
    mesh=_MESH,
    out_type=jax.ShapeDtypeStruct((2, N, 128), jnp.float32),
    scratch_types=[
        pltpu.VMEM((CHUNK,), jnp.int32),
        pltpu.VMEM((CHUNK,), jnp.int32),
        pltpu.VMEM((CHUNK,), jnp.int32),
        pltpu.VMEM((CHUNK,), jnp.int32),
        pltpu.VMEM((CHUNK, 128), jnp.float32),
        pltpu.VMEM((CHUNK, 128), jnp.float32),
        pltpu.VMEM_SHARED((N, 128), jnp.float32),
        pltpu.SemaphoreType.DMA,
        pltpu.SemaphoreType.DMA,
        pltpu.SemaphoreType.DMA,
        pltpu.SemaphoreType.DMA,
        pltpu.SemaphoreType.DMA,
        pltpu.SemaphoreType.DMA,
    ],
)
def _sc_prop(g_hbm, src_hbm, dst_hbm, zrows_hbm, acc_out,
             src_a, dst_a, src_b, dst_b, rows_a, rows_b, acc_sh,
             sem_sa, sem_da, sem_sb, sem_db, sem_ga, sem_gb):
    cid = lax.axis_index("c")
    sid = lax.axis_index("s")
    wid = sid * 2 + cid

    def idx_start(k, s_ref, d_ref, s_sem, d_sem):
        base = (wid + k * NW) * CHUNK
        pltpu.async_copy(src_hbm.at[pl.ds(base, CHUNK)], s_ref, s_sem)
        pltpu.async_copy(dst_hbm.at[pl.ds(base, CHUNK)], d_ref, d_sem)

    def idx_wait(s_ref, d_ref, s_sem, d_sem):
        pltpu.make_async_copy(src_hbm.at[pl.ds(0, CHUNK)], s_ref, s_sem).wait()
        pltpu.make_async_copy(dst_hbm.at[pl.ds(0, CHUNK)], d_ref, d_sem).wait()

    def gather_start(s_ref, rows, gsem):
        pltpu.async_copy(g_hbm.at[s_ref], rows, gsem)

    def gather_wait(s_ref, rows, gsem):
        pltpu.make_async_copy(g_hbm.at[s_ref], rows, gsem).wait()

    pltpu.sync_copy(zrows_hbm, rows_a)

    @pl.when(sid < 15)
    def _():
        row0 = sid * ROWS_LO
        for off, size in ((0, 128), (128, 128), (256, 128), (384, 128),
                          (512, 112)):
            pltpu.sync_copy(rows_a.at[pl.ds(0, size)],
                            acc_sh.at[pl.ds(row0 + off, size)])

    @pl.when(sid == 15)
    def _():
        for off in (0, 128, 256, 384, 512):
            pltpu.sync_copy(rows_a, acc_sh.at[pl.ds(ROW_SPLIT + off, 128)])

    plsc.subcore_barrier()

    idx_start(0, src_a, dst_a, sem_sa, sem_da)
    idx_start(1, src_b, dst_b, sem_sb, sem_db)
    idx_wait(src_a, dst_a, sem_sa, sem_da)
    gather_start(src_a, rows_a, sem_ga)

    def body(j, _):
        k2 = jnp.minimum(2 * j + 2, T_MAIN - 1)
        k3 = jnp.minimum(2 * j + 3, T_MAIN - 1)
        idx_wait(src_b, dst_b, sem_sb, sem_db)
        gather_wait(src_a, rows_a, sem_ga)
        gather_start(src_b, rows_b, sem_gb)
        pltpu.sync_copy(rows_a, acc_sh.at[dst_a], add=True)
        idx_start(k2, src_a, dst_a, sem_sa, sem_da)
        idx_wait(src_a, dst_a, sem_sa, sem_da)
        gather_wait(src_b, rows_b, sem_gb)
        gather_start(src_a, rows_a, sem_ga)
        pltpu.sync_copy(rows_b, acc_sh.at[dst_b], add=True)
        idx_start(k3, src_b, dst_b, sem_sb, sem_db)
        return 0

    lax.fori_loop(0, T_MAIN // 2, body, 0)
    gather_wait(src_a, rows_a, sem_ga)
    idx_wait(src_b, dst_b, sem_sb, sem_db)

    @pl.when(wid < NTAIL)
    def _():
        base = (NFULL + wid) * CHUNK
        pltpu.sync_copy(src_hbm.at[pl.ds(base, CHUNK)], src_a)
        pltpu.sync_copy(dst_hbm.at[pl.ds(base, CHUNK)], dst_a)
        pltpu.async_copy(g_hbm.at[src_a], rows_a, sem_ga).wait()
        pltpu.sync_copy(rows_a, acc_sh.at[dst_a], add=True)

    plsc.subcore_barrier()

    @pl.when(sid < 15)
    def _():
        pltpu.sync_copy(acc_sh.at[pl.ds(sid * ROWS_LO, ROWS_LO)],
                        acc_out.at[cid].at[pl.ds(sid * ROWS_LO, ROWS_LO)])

    @pl.when(sid == 15)
    def _():
        pltpu.sync_copy(acc_sh.at[pl.ds(ROW_SPLIT, ROWS_HI)],
                        acc_out.at[cid].at[pl.ds(ROW_SPLIT, ROWS_HI)])


_BLK = 2000


def _row_spec(width):
    return pl.BlockSpec((_BLK, width), lambda i: (i, 0))


def _full_spec(shape):
    nd = len(shape)
    return pl.BlockSpec(shape, lambda i: (0,) * nd)


def _tc_call(body, out_shapes, in_specs, out_specs):
    return pl.pallas_call(
        body,
        grid=(N // _BLK,),
        in_specs=in_specs,
        out_specs=out_specs,
        out_shape=out_shapes,
    )


def _k1_body(degp_ref, x_ref, w_ref, dinv_ref, g_ref):
    deg = degp_ref[0] + degp_ref[1] + 1.0
    dinv = lax.rsqrt(deg)
    dinv_ref[...] = dinv
    g_ref[...] = dinv * jnp.dot(x_ref[...], w_ref[...],
                                preferred_element_type=jnp.float32)


def _k2_body(ap_ref, g_ref, dinv_ref, b1_ref, wml_ref, g2_ref):
    dinv = dinv_ref[...]
    h = jnp.tanh(dinv * (ap_ref[0] + ap_ref[1] + g_ref[...]) + b1_ref[...])
    g2_ref[...] = dinv * jnp.dot(h, wml_ref[...],
                                 preferred_element_type=jnp.float32)


def _k3_body(ap_ref, g_ref, dinv_ref, bm_ref, bl_ref, noise_ref, wd1_ref,
             mean_ref, logvar_ref, z_ref, g3_ref):
    dinv = dinv_ref[...]
    t = dinv * (ap_ref[0] + ap_ref[1] + g_ref[...])
    mean = t[:, :64] + bm_ref[...]
    logvar = t[:, 64:] + bl_ref[...]
    z = noise_ref[...] * jnp.exp(0.5 * logvar) + mean
    mean_ref[...] = mean
    logvar_ref[...] = logvar
    z_ref[...] = z
    g3_ref[...] = dinv * jnp.dot(z, wd1_ref[...],
                                 preferred_element_type=jnp.float32)


def _k4_body(ap_ref, g_ref, dinv_ref, bd1_ref, wd2_ref, g4_ref):
    dinv = dinv_ref[...]
    hd = jnp.tanh(dinv * (ap_ref[0] + ap_ref[1] + g_ref[...]) + bd1_ref[...])
    g4_ref[...] = dinv * jnp.dot(hd, wd2_ref[...],
                                 preferred_element_type=jnp.float32)


def _k5_body(ap_ref, g_ref, dinv_ref, bd2_ref, out_ref):
    out_ref[...] = (dinv_ref[...] * (ap_ref[0] + ap_ref[1] + g_ref[...])
                    + bd2_ref[...])


def _pair_spec(width):
    return pl.BlockSpec((2, _BLK, width), lambda i: (0, i, 0))


def kernel(feature, edge_index, W1, b1, Wm, bm, Wl, bl, Wd1, bd1, Wd2, bd2):
    f32 = jnp.float32
    src = edge_index[0]
    dst = edge_index[1]

    ones_col = jnp.ones((CHUNK,), f32)
    zcol = jnp.zeros((D1_LO,), f32)
    zrows = jnp.zeros((CHUNK, 128), f32)
    noise = jax.random.normal(jax.random.key(42), (N, 64), dtype=f32)
    Wml = jnp.concatenate([Wm, Wl], axis=1)

    deg_parts = _sc_degree(dst, ones_col, zcol)[:, :N, None]

    dinv, g1 = _tc_call(
        _k1_body,
        [jax.ShapeDtypeStruct((N, 1), f32), jax.ShapeDtypeStruct((N, 128), f32)],
        [_pair_spec(1), _row_spec(128), _full_spec((128, 128))],
        [_row_spec(1), _row_spec(128)],
    )(deg_parts, feature, W1)

    acc1 = _sc_prop(g1, src, dst, zrows)

    g2 = _tc_call(
        _k2_body,
        jax.ShapeDtypeStruct((N, 128), f32),
        [_pair_spec(128), _row_spec(128), _row_spec(1), _full_spec((128,)),
         _full_spec((128, 128))],
        _row_spec(128),
    )(acc1, g1, dinv, b1, Wml)

    acc2 = _sc_prop(g2, src, dst, zrows)

    mean, logvar, z, g3 = _tc_call(
        _k3_body,
        [jax.ShapeDtypeStruct((N, 64), f32), jax.ShapeDtypeStruct((N, 64), f32),
         jax.ShapeDtypeStruct((N, 64), f32), jax.ShapeDtypeStruct((N, 128), f32)],
        [_pair_spec(128), _row_spec(128), _row_spec(1), _full_spec((64,)),
         _full_spec((64,)), _row_spec(64), _full_spec((64, 128))],
        [_row_spec(64), _row_spec(64), _row_spec(64), _row_spec(128)],
    )(acc2, g2, dinv, bm, bl, noise, Wd1)

    acc3 = _sc_prop(g3, src, dst, zrows)

    g4 = _tc_call(
        _k4_body,
        jax.ShapeDtypeStruct((N, 128), f32),
        [_pair_spec(128), _row_spec(128), _row_spec(1), _full_spec((128,)),
         _full_spec((128, 128))],
        _row_spec(128),
    )(acc3, g3, dinv, bd1, Wd2)

    acc4 = _sc_prop(g4, src, dst, zrows)

    out = _tc_call(
        _k5_body,
        jax.ShapeDtypeStruct((N, 128), f32),
        [_pair_spec(128), _row_spec(128), _row_spec(1), _full_spec((128,))],
        _row_spec(128),
    )(acc4, g4, dinv, bd2)

    return (z, mean, logvar, out)


# speedup vs baseline: 28.7859x; 1.1537x over previous
import functools

import jax
import jax.numpy as jnp
from jax import lax
from jax.experimental import pallas as pl
from jax.experimental.pallas import tpu as pltpu
from jax.experimental.pallas import tpu_sc as plsc

N = 10000
E = 320000
CHUNK = 128
NCHUNKS = E // CHUNK
NW = 32
ROWS_LO = 624
ROWS_HI = 640
ROW_SPLIT = 15 * ROWS_LO
D1_LO = 640
NPAD1 = 16 * D1_LO

_MESH = plsc.VectorSubcoreMesh(core_axis_name="c", subcore_axis_name="s")


@functools.partial(
    pl.kernel,
    mesh=_MESH,
    out_type=jax.ShapeDtypeStruct((2, NPAD1), jnp.float32),
    scratch_types=[
        pltpu.VMEM((CHUNK,), jnp.int32),
        pltpu.VMEM((CHUNK,), jnp.float32),
        pltpu.VMEM((D1_LO,), jnp.float32),
        pltpu.VMEM_SHARED((NPAD1,), jnp.float32),
    ],
)
def _sc_degree(dst_hbm, ones_hbm, zcol_hbm, deg_out, idx_v, ones_v, z_v, acc_sh):
    cid = lax.axis_index("c")
    sid = lax.axis_index("s")
    wid = sid * 2 + cid

    pltpu.sync_copy(ones_hbm, ones_v)
    pltpu.sync_copy(zcol_hbm, z_v)
    pltpu.sync_copy(z_v, acc_sh.at[pl.ds(sid * D1_LO, D1_LO)])
    plsc.subcore_barrier()

    n_i = (NCHUNKS - wid + NW - 1) // NW

    def body(i, _):
        base = (wid + i * NW) * CHUNK
        pltpu.sync_copy(dst_hbm.at[pl.ds(base, CHUNK)], idx_v)
        pltpu.sync_copy(ones_v, acc_sh.at[idx_v], add=True)
        return 0

    lax.fori_loop(0, n_i, body, 0)
    plsc.subcore_barrier()

    pltpu.sync_copy(acc_sh.at[pl.ds(sid * D1_LO, D1_LO)],
                    deg_out.at[cid].at[pl.ds(sid * D1_LO, D1_LO)])


NFULL = (NCHUNKS // NW) * NW
NTAIL = NCHUNKS - NFULL
T_MAIN = NFULL // NW


@functools.partial(
    pl.kernel,
    mesh=_MESH,
    out_type=jax.ShapeDtypeStruct((2, N, 128), jnp.float32),
    scratch_types=[
        pltpu.VMEM((3, CHUNK), jnp.int32),
        pltpu.VMEM((3, CHUNK), jnp.int32),
        pltpu.VMEM((CHUNK, 128), jnp.float32),
        pltpu.VMEM((CHUNK, 128), jnp.float32),
        pltpu.VMEM((CHUNK, 128), jnp.float32),
        pltpu.VMEM_SHARED((N, 128), jnp.float32),
        pltpu.SemaphoreType.DMA,
        pltpu.SemaphoreType.DMA,
        pltpu.SemaphoreType.DMA,
        pltpu.SemaphoreType.DMA,
        pltpu.SemaphoreType.DMA,
        pltpu.SemaphoreType.DMA,
        pltpu.SemaphoreType.DMA,
        pltpu.SemaphoreType.DMA,
        pltpu.SemaphoreType.DMA,
    ],
)
def _sc_prop(g_hbm, src_hbm, dst_hbm, zrows_hbm, acc_out,
             src_v, dst_v, rows_0, rows_1, rows_2, acc_sh,
             sem_s0, sem_s1, sem_s2, sem_d0, sem_d1, sem_d2,
             sem_g0, sem_g1, sem_g2):
    cid = lax.axis_index("c")
    sid = lax.axis_index("s")
    wid = sid * 2 + cid

    rows = (rows_0, rows_1, rows_2)
    sem_s = (sem_s0, sem_s1, sem_s2)
    sem_d = (sem_d0, sem_d1, sem_d2)
    sem_g = (sem_g0, sem_g1, sem_g2)

    def idx_start(k, m):
        base = (wid + k * NW) * CHUNK
        pltpu.async_copy(src_hbm.at[pl.ds(base, CHUNK)], src_v.at[m], sem_s[m])
        pltpu.async_copy(dst_hbm.at[pl.ds(base, CHUNK)], dst_v.at[m], sem_d[m])

    def idx_wait(m):
        pltpu.make_async_copy(src_hbm.at[pl.ds(0, CHUNK)], src_v.at[m],
                              sem_s[m]).wait()
        pltpu.make_async_copy(dst_hbm.at[pl.ds(0, CHUNK)], dst_v.at[m],
                              sem_d[m]).wait()

    def gather_start(m):
        pltpu.async_copy(g_hbm.at[src_v.at[m]], rows[m], sem_g[m])

    def gather_wait(m):
        pltpu.make_async_copy(g_hbm.at[src_v.at[m]], rows[m], sem_g[m]).wait()

    pltpu.sync_copy(zrows_hbm, rows_0)

    @pl.when(sid < 15)
    def _():
        row0 = sid * ROWS_LO
        for off, size in ((0, 128), (128, 128), (256, 128), (384, 128),
                          (512, 112)):
            pltpu.sync_copy(rows_0.at[pl.ds(0, size)],
                            acc_sh.at[pl.ds(row0 + off, size)])

    @pl.when(sid == 15)
    def _():
        for off in (0, 128, 256, 384, 512):
            pltpu.sync_copy(rows_0, acc_sh.at[pl.ds(ROW_SPLIT + off, 128)])

    plsc.subcore_barrier()

    idx_start(0, 0)
    idx_start(1, 1)
    idx_start(2, 2)
    idx_wait(0)
    gather_start(0)
    idx_wait(1)
    gather_start(1)

    def phase(k, m):
        gather_wait(m)
        pltpu.sync_copy(rows[m], acc_sh.at[dst_v.at[m]], add=True)
        idx_start(jnp.minimum(k + 3, T_MAIN - 1), m)
        m2 = (m + 2) % 3
        idx_wait(m2)
        gather_start(m2)

    def body(j, _):
        k = 3 * j
        phase(k, 0)
        phase(k + 1, 1)
        phase(k + 2, 2)
        return 0

    lax.fori_loop(0, T_MAIN // 3, body, 0)
    gather_wait(0)
    gather_wait(1)
    idx_wait(2)

    @pl.when(wid < NTAIL)
    def _():
        base = (NFULL + wid) * CHUNK
        pltpu.sync_copy(src_hbm.at[pl.ds(base, CHUNK)], src_v.at[0])
        pltpu.sync_copy(dst_hbm.at[pl.ds(base, CHUNK)], dst_v.at[0])
        pltpu.async_copy(g_hbm.at[src_v.at[0]], rows_0, sem_g0).wait()
        pltpu.sync_copy(rows_0, acc_sh.at[dst_v.at[0]], add=True)

    plsc.subcore_barrier()

    @pl.when(sid < 15)
    def _():
        pltpu.sync_copy(acc_sh.at[pl.ds(sid * ROWS_LO, ROWS_LO)],
                        acc_out.at[cid].at[pl.ds(sid * ROWS_LO, ROWS_LO)])

    @pl.when(sid == 15)
    def _():
        pltpu.sync_copy(acc_sh.at[pl.ds(ROW_SPLIT, ROWS_HI)],
                        acc_out.at[cid].at[pl.ds(ROW_SPLIT, ROWS_HI)])


_BLK = 2000


def _row_spec(width):
    return pl.BlockSpec((_BLK, width), lambda i: (i, 0))


def _full_spec(shape):
    nd = len(shape)
    return pl.BlockSpec(shape, lambda i: (0,) * nd)


def _tc_call(body, out_shapes, in_specs, out_specs):
    return pl.pallas_call(
        body,
        grid=(N // _BLK,),
        in_specs=in_specs,
        out_specs=out_specs,
        out_shape=out_shapes,
    )


def _k1_body(degp_ref, x_ref, w_ref, dinv_ref, g_ref):
    deg = degp_ref[0] + degp_ref[1] + 1.0
    dinv = lax.rsqrt(deg)
    dinv_ref[...] = dinv
    g_ref[...] = dinv * jnp.dot(x_ref[...], w_ref[...],
                                preferred_element_type=jnp.float32)


def _k2_body(ap_ref, g_ref, dinv_ref, b1_ref, wml_ref, g2_ref):
    dinv = dinv_ref[...]
    h = jnp.tanh(dinv * (ap_ref[0] + ap_ref[1] + g_ref[...]) + b1_ref[...])
    g2_ref[...] = dinv * jnp.dot(h, wml_ref[...],
                                 preferred_element_type=jnp.float32)


def _k3_body(ap_ref, g_ref, dinv_ref, bm_ref, bl_ref, noise_ref, wd1_ref,
             mean_ref, logvar_ref, z_ref, g3_ref):
    dinv = dinv_ref[...]
    t = dinv * (ap_ref[0] + ap_ref[1] + g_ref[...])
    mean = t[:, :64] + bm_ref[...]
    logvar = t[:, 64:] + bl_ref[...]
    z = noise_ref[...] * jnp.exp(0.5 * logvar) + mean
    mean_ref[...] = mean
    logvar_ref[...] = logvar
    z_ref[...] = z
    g3_ref[...] = dinv * jnp.dot(z, wd1_ref[...],
                                 preferred_element_type=jnp.float32)


def _k4_body(ap_ref, g_ref, dinv_ref, bd1_ref, wd2_ref, g4_ref):
    dinv = dinv_ref[...]
    hd = jnp.tanh(dinv * (ap_ref[0] + ap_ref[1] + g_ref[...]) + bd1_ref[...])
    g4_ref[...] = dinv * jnp.dot(hd, wd2_ref[...],
                                 preferred_element_type=jnp.float32)


def _k5_body(ap_ref, g_ref, dinv_ref, bd2_ref, out_ref):
    out_ref[...] = (dinv_ref[...] * (ap_ref[0] + ap_ref[1] + g_ref[...])
                    + bd2_ref[...])


def _pair_spec(width):
    return pl.BlockSpec((2, _BLK, width), lambda i: (0, i, 0))


def kernel(feature, edge_index, W1, b1, Wm, bm, Wl, bl, Wd1, bd1, Wd2, bd2):
    f32 = jnp.float32
    src = edge_index[0]
    dst = edge_index[1]

    ones_col = jnp.ones((CHUNK,), f32)
    zcol = jnp.zeros((D1_LO,), f32)
    zrows = jnp.zeros((CHUNK, 128), f32)
    noise = jax.random.normal(jax.random.key(42), (N, 64), dtype=f32)
    Wml = jnp.concatenate([Wm, Wl], axis=1)

    deg_parts = _sc_degree(dst, ones_col, zcol)[:, :N, None]

    dinv, g1 = _tc_call(
        _k1_body,
        [jax.ShapeDtypeStruct((N, 1), f32), jax.ShapeDtypeStruct((N, 128), f32)],
        [_pair_spec(1), _row_spec(128), _full_spec((128, 128))],
        [_row_spec(1), _row_spec(128)],
    )(deg_parts, feature, W1)

    acc1 = _sc_prop(g1, src, dst, zrows)

    g2 = _tc_call(
        _k2_body,
        jax.ShapeDtypeStruct((N, 128), f32),
        [_pair_spec(128), _row_spec(128), _row_spec(1), _full_spec((128,)),
         _full_spec((128, 128))],
        _row_spec(128),
    )(acc1, g1, dinv, b1, Wml)

    acc2 = _sc_prop(g2, src, dst, zrows)

    mean, logvar, z, g3 = _tc_call(
        _k3_body,
        [jax.ShapeDtypeStruct((N, 64), f32), jax.ShapeDtypeStruct((N, 64), f32),
         jax.ShapeDtypeStruct((N, 64), f32), jax.ShapeDtypeStruct((N, 128), f32)],
        [_pair_spec(128), _row_spec(128), _row_spec(1), _full_spec((64,)),
         _full_spec((64,)), _row_spec(64), _full_spec((64, 128))],
        [_row_spec(64), _row_spec(64), _row_spec(64), _row_spec(128)],
    )(acc2, g2, dinv, bm, bl, noise, Wd1)

    acc3 = _sc_prop(g3, src, dst, zrows)

    g4 = _tc_call(
        _k4_body,
        jax.ShapeDtypeStruct((N, 128), f32),
        [_pair_spec(128), _row_spec(128), _row_spec(1), _full_spec((128,)),
         _full_spec((128, 128))],
        _row_spec(128),
    )(acc3, g3, dinv, bd1, Wd2)

    acc4 = _sc_prop(g4, src, dst, zrows)

    out = _tc_call(
        _k5_body,
        jax.ShapeDtypeStruct((N, 128), f32),
        [_pair_spec(128), _row_spec(128), _row_spec(1), _full_spec((128,))],
        _row_spec(128),
    )(acc4, g4, dinv, bd2)

    return (z, mean, logvar, out)


# speedup vs baseline: 30.1999x; 1.0491x over previous
import functools

import jax
import jax.numpy as jnp
from jax import lax
from jax.experimental import pallas as pl
from jax.experimental.pallas import tpu as pltpu
from jax.experimental.pallas import tpu_sc as plsc

N = 10000
E = 320000
CHUNK = 128
NCHUNKS = E // CHUNK
NW = 32
ROWS_LO = 624
ROWS_HI = 640
ROW_SPLIT = 15 * ROWS_LO
D1_LO = 640
NPAD1 = 16 * D1_LO

_MESH = plsc.VectorSubcoreMesh(core_axis_name="c", subcore_axis_name="s")

NFULL = (NCHUNKS // NW) * NW
NTAIL = NCHUNKS - NFULL
T_MAIN = NFULL // NW


@functools.partial(
    pl.kernel,
    mesh=_MESH,
    out_type=jax.ShapeDtypeStruct((2, NPAD1), jnp.float32),
    scratch_types=[
        pltpu.VMEM((3, CHUNK), jnp.int32),
        pltpu.VMEM((CHUNK,), jnp.float32),
        pltpu.VMEM((D1_LO,), jnp.float32),
        pltpu.VMEM_SHARED((NPAD1,), jnp.float32),
        pltpu.SemaphoreType.DMA,
        pltpu.SemaphoreType.DMA,
        pltpu.SemaphoreType.DMA,
    ],
)
def _sc_degree(dst_hbm, ones_hbm, zcol_hbm, deg_out, idx_v, ones_v, z_v,
               acc_sh, sem_0, sem_1, sem_2):
    cid = lax.axis_index("c")
    sid = lax.axis_index("s")
    wid = sid * 2 + cid
    sems = (sem_0, sem_1, sem_2)

    def idx_start(k, m):
        base = (wid + k * NW) * CHUNK
        pltpu.async_copy(dst_hbm.at[pl.ds(base, CHUNK)], idx_v.at[m], sems[m])

    def idx_wait(m):
        pltpu.make_async_copy(dst_hbm.at[pl.ds(0, CHUNK)], idx_v.at[m],
                              sems[m]).wait()

    pltpu.sync_copy(ones_hbm, ones_v)
    pltpu.sync_copy(zcol_hbm, z_v)
    pltpu.sync_copy(z_v, acc_sh.at[pl.ds(sid * D1_LO, D1_LO)])
    plsc.subcore_barrier()

    for m in range(3):
        idx_start(m, m)

    def phase(k, m):
        idx_wait(m)
        pltpu.sync_copy(ones_v, acc_sh.at[idx_v.at[m]], add=True)
        idx_start(jnp.minimum(k + 3, T_MAIN - 1), m)

    def body(j, _):
        k = 3 * j
        for m in range(3):
            phase(k + m, m)
        return 0

    lax.fori_loop(0, T_MAIN // 3, body, 0)
    for m in range(3):
        idx_wait(m)

    @pl.when(wid < NTAIL)
    def _():
        base = (NFULL + wid) * CHUNK
        pltpu.sync_copy(dst_hbm.at[pl.ds(base, CHUNK)], idx_v.at[0])
        pltpu.sync_copy(ones_v, acc_sh.at[idx_v.at[0]], add=True)

    plsc.subcore_barrier()

    pltpu.sync_copy(acc_sh.at[pl.ds(sid * D1_LO, D1_LO)],
                    deg_out.at[cid].at[pl.ds(sid * D1_LO, D1_LO)])


@functools.partial(
    pl.kernel,
    mesh=_MESH,
    out_type=jax.ShapeDtypeStruct((2, N, 128), jnp.float32),
    scratch_types=[
        pltpu.VMEM((3, CHUNK), jnp.int32),
        pltpu.VMEM((3, CHUNK), jnp.int32),
        pltpu.VMEM((CHUNK, 128), jnp.float32),
        pltpu.VMEM((CHUNK, 128), jnp.float32),
        pltpu.VMEM((CHUNK, 128), jnp.float32),
        pltpu.VMEM_SHARED((N, 128), jnp.float32),
        pltpu.SemaphoreType.DMA,
        pltpu.SemaphoreType.DMA,
        pltpu.SemaphoreType.DMA,
        pltpu.SemaphoreType.DMA,
        pltpu.SemaphoreType.DMA,
        pltpu.SemaphoreType.DMA,
        pltpu.SemaphoreType.DMA,
        pltpu.SemaphoreType.DMA,
        pltpu.SemaphoreType.DMA,
    ],
)
def _sc_prop(g_hbm, src_hbm, dst_hbm, zrows_hbm, acc_out,
             src_v, dst_v, rows_0, rows_1, rows_2, acc_sh,
             sem_s0, sem_s1, sem_s2, sem_d0, sem_d1, sem_d2,
             sem_g0, sem_g1, sem_g2):
    cid = lax.axis_index("c")
    sid = lax.axis_index("s")
    wid = sid * 2 + cid

    rows = (rows_0, rows_1, rows_2)
    sem_s = (sem_s0, sem_s1, sem_s2)
    sem_d = (sem_d0, sem_d1, sem_d2)
    sem_g = (sem_g0, sem_g1, sem_g2)

    def idx_start(k, m):
        base = (wid + k * NW) * CHUNK
        pltpu.async_copy(src_hbm.at[pl.ds(base, CHUNK)], src_v.at[m], sem_s[m])
        pltpu.async_copy(dst_hbm.at[pl.ds(base, CHUNK)], dst_v.at[m], sem_d[m])

    def idx_wait(m):
        pltpu.make_async_copy(src_hbm.at[pl.ds(0, CHUNK)], src_v.at[m],
                              sem_s[m]).wait()
        pltpu.make_async_copy(dst_hbm.at[pl.ds(0, CHUNK)], dst_v.at[m],
                              sem_d[m]).wait()

    def gather_start(m):
        pltpu.async_copy(g_hbm.at[src_v.at[m]], rows[m], sem_g[m])

    def gather_wait(m):
        pltpu.make_async_copy(g_hbm.at[src_v.at[m]], rows[m], sem_g[m]).wait()

    pltpu.sync_copy(zrows_hbm, rows_0)

    @pl.when(sid < 15)
    def _():
        row0 = sid * ROWS_LO
        for off, size in ((0, 128), (128, 128), (256, 128), (384, 128),
                          (512, 112)):
            pltpu.sync_copy(rows_0.at[pl.ds(0, size)],
                            acc_sh.at[pl.ds(row0 + off, size)])

    @pl.when(sid == 15)
    def _():
        for off in (0, 128, 256, 384, 512):
            pltpu.sync_copy(rows_0, acc_sh.at[pl.ds(ROW_SPLIT + off, 128)])

    plsc.subcore_barrier()

    for m in range(3):
        idx_start(m, m)
    for m in range(2):
        idx_wait(m)
        gather_start(m)

    def phase(k, m):
        gather_wait(m)
        pltpu.sync_copy(rows[m], acc_sh.at[dst_v.at[m]], add=True)
        idx_start(jnp.minimum(k + 3, T_MAIN - 1), m)
        m2 = (m + 2) % 3
        idx_wait(m2)
        gather_start(m2)

    def body(j, _):
        k = 3 * j
        for m in range(3):
            phase(k + m, m)
        return 0

    lax.fori_loop(0, T_MAIN // 3, body, 0)
    gather_wait(0)
    gather_wait(1)
    idx_wait(2)

    @pl.when(wid < NTAIL)
    def _():
        base = (NFULL + wid) * CHUNK
        pltpu.sync_copy(src_hbm.at[pl.ds(base, CHUNK)], src_v.at[0])
        pltpu.sync_copy(dst_hbm.at[pl.ds(base, CHUNK)], dst_v.at[0])
        pltpu.async_copy(g_hbm.at[src_v.at[0]], rows_0, sem_g0).wait()
        pltpu.sync_copy(rows_0, acc_sh.at[dst_v.at[0]], add=True)

    plsc.subcore_barrier()

    @pl.when(sid < 15)
    def _():
        pltpu.sync_copy(acc_sh.at[pl.ds(sid * ROWS_LO, ROWS_LO)],
                        acc_out.at[cid].at[pl.ds(sid * ROWS_LO, ROWS_LO)])

    @pl.when(sid == 15)
    def _():
        pltpu.sync_copy(acc_sh.at[pl.ds(ROW_SPLIT, ROWS_HI)],
                        acc_out.at[cid].at[pl.ds(ROW_SPLIT, ROWS_HI)])


_BLK = 2000


def _row_spec(width):
    return pl.BlockSpec((_BLK, width), lambda i: (i, 0))


def _full_spec(shape):
    nd = len(shape)
    return pl.BlockSpec(shape, lambda i: (0,) * nd)


def _tc_call(body, out_shapes, in_specs, out_specs):
    return pl.pallas_call(
        body,
        grid=(N // _BLK,),
        in_specs=in_specs,
        out_specs=out_specs,
        out_shape=out_shapes,
    )


def _k0_body(x_ref, w_ref, h_ref):
    h_ref[...] = jnp.dot(x_ref[...], w_ref[...],
                         preferred_element_type=jnp.float32)


def _k1_body(degp_ref, h_ref, dinv_ref, g_ref):
    deg = degp_ref[0] + degp_ref[1] + 1.0
    dinv = lax.rsqrt(deg)
    dinv_ref[...] = dinv
    g_ref[...] = dinv * h_ref[...]


def _k2_body(ap_ref, g_ref, dinv_ref, b1_ref, wml_ref, g2_ref):
    dinv = dinv_ref[...]
    h = jnp.tanh(dinv * (ap_ref[0] + ap_ref[1] + g_ref[...]) + b1_ref[...])
    g2_ref[...] = dinv * jnp.dot(h, wml_ref[...],
                                 preferred_element_type=jnp.float32)


def _k3_body(ap_ref, g_ref, dinv_ref, bm_ref, bl_ref, noise_ref, wd1_ref,
             mean_ref, logvar_ref, z_ref, g3_ref):
    dinv = dinv_ref[...]
    t = dinv * (ap_ref[0] + ap_ref[1] + g_ref[...])
    mean = t[:, :64] + bm_ref[...]
    logvar = t[:, 64:] + bl_ref[...]
    z = noise_ref[...] * jnp.exp(0.5 * logvar) + mean
    mean_ref[...] = mean
    logvar_ref[...] = logvar
    z_ref[...] = z
    g3_ref[...] = dinv * jnp.dot(z, wd1_ref[...],
                                 preferred_element_type=jnp.float32)


def _k4_body(ap_ref, g_ref, dinv_ref, bd1_ref, wd2_ref, g4_ref):
    dinv = dinv_ref[...]
    hd = jnp.tanh(dinv * (ap_ref[0] + ap_ref[1] + g_ref[...]) + bd1_ref[...])
    g4_ref[...] = dinv * jnp.dot(hd, wd2_ref[...],
                                 preferred_element_type=jnp.float32)


def _k5_body(ap_ref, g_ref, dinv_ref, bd2_ref, out_ref):
    out_ref[...] = (dinv_ref[...] * (ap_ref[0] + ap_ref[1] + g_ref[...])
                    + bd2_ref[...])


def _pair_spec(width):
    return pl.BlockSpec((2, _BLK, width), lambda i: (0, i, 0))


def kernel(feature, edge_index, W1, b1, Wm, bm, Wl, bl, Wd1, bd1, Wd2, bd2):
    f32 = jnp.float32
    src = edge_index[0]
    dst = edge_index[1]

    ones_col = jnp.ones((CHUNK,), f32)
    zcol = jnp.zeros((D1_LO,), f32)
    zrows = jnp.zeros((CHUNK, 128), f32)
    noise = jax.random.normal(jax.random.key(42), (N, 64), dtype=f32)
    Wml = jnp.concatenate([Wm, Wl], axis=1)

    h1 = _tc_call(
        _k0_body,
        jax.ShapeDtypeStruct((N, 128), f32),
        [_row_spec(128), _full_spec((128, 128))],
        _row_spec(128),
    )(feature, W1)

    deg_parts = _sc_degree(dst, ones_col, zcol)[:, :N, None]

    dinv, g1 = _tc_call(
        _k1_body,
        [jax.ShapeDtypeStruct((N, 1), f32), jax.ShapeDtypeStruct((N, 128), f32)],
        [_pair_spec(1), _row_spec(128)],
        [_row_spec(1), _row_spec(128)],
    )(deg_parts, h1)

    acc1 = _sc_prop(g1, src, dst, zrows)

    g2 = _tc_call(
        _k2_body,
        jax.ShapeDtypeStruct((N, 128), f32),
        [_pair_spec(128), _row_spec(128), _row_spec(1), _full_spec((128,)),
         _full_spec((128, 128))],
        _row_spec(128),
    )(acc1, g1, dinv, b1, Wml)

    acc2 = _sc_prop(g2, src, dst, zrows)

    mean, logvar, z, g3 = _tc_call(
        _k3_body,
        [jax.ShapeDtypeStruct((N, 64), f32), jax.ShapeDtypeStruct((N, 64), f32),
         jax.ShapeDtypeStruct((N, 64), f32), jax.ShapeDtypeStruct((N, 128), f32)],
        [_pair_spec(128), _row_spec(128), _row_spec(1), _full_spec((64,)),
         _full_spec((64,)), _row_spec(64), _full_spec((64, 128))],
        [_row_spec(64), _row_spec(64), _row_spec(64), _row_spec(128)],
    )(acc2, g2, dinv, bm, bl, noise, Wd1)

    acc3 = _sc_prop(g3, src, dst, zrows)

    g4 = _tc_call(
        _k4_body,
        jax.ShapeDtypeStruct((N, 128), f32),
        [_pair_spec(128), _row_spec(128), _row_spec(1), _full_spec((128,)),
         _full_spec((128, 128))],
        _row_spec(128),
    )(acc3, g3, dinv, bd1, Wd2)

    acc4 = _sc_prop(g4, src, dst, zrows)

    out = _tc_call(
        _k5_body,
        jax.ShapeDtypeStruct((N, 128), f32),
        [_pair_spec(128), _row_spec(128), _row_spec(1), _full_spec((128,))],
        _row_spec(128),
    )(acc4, g4, dinv, bd2)

    return (z, mean, logvar, out)


# speedup vs baseline: 31.5412x; 1.0444x over previous
import functools

import jax
import jax.numpy as jnp
from jax import lax
from jax.experimental import pallas as pl
from jax.experimental.pallas import tpu as pltpu
from jax.experimental.pallas import tpu_sc as plsc

N = 10000
E = 320000
CHUNK = 128
NCHUNKS = E // CHUNK
NW = 32
ROWS_LO = 624
ROWS_HI = 640
ROW_SPLIT = 15 * ROWS_LO
D1_LO = 640
NPAD1 = 16 * D1_LO

_MESH = plsc.VectorSubcoreMesh(core_axis_name="c", subcore_axis_name="s")

NFULL = (NCHUNKS // NW) * NW
NTAIL = NCHUNKS - NFULL
T_MAIN = NFULL // NW


@functools.partial(
    pl.kernel,
    mesh=_MESH,
    out_type=jax.ShapeDtypeStruct((2, NPAD1), jnp.float32),
    scratch_types=[
        pltpu.VMEM((3, CHUNK), jnp.int32),
        pltpu.VMEM((CHUNK,), jnp.float32),
        pltpu.VMEM((D1_LO,), jnp.float32),
        pltpu.VMEM_SHARED((NPAD1,), jnp.float32),
        pltpu.SemaphoreType.DMA,
        pltpu.SemaphoreType.DMA,
        pltpu.SemaphoreType.DMA,
    ],
)
def _sc_degree(dst_hbm, ones_hbm, zcol_hbm, deg_out, idx_v, ones_v, z_v,
               acc_sh, sem_0, sem_1, sem_2):
    cid = lax.axis_index("c")
    sid = lax.axis_index("s")
    wid = sid * 2 + cid
    sems = (sem_0, sem_1, sem_2)

    def idx_start(k, m):
        base = (wid + k * NW) * CHUNK
        pltpu.async_copy(dst_hbm.at[pl.ds(base, CHUNK)], idx_v.at[m], sems[m])

    def idx_wait(m):
        pltpu.make_async_copy(dst_hbm.at[pl.ds(0, CHUNK)], idx_v.at[m],
                              sems[m]).wait()

    pltpu.sync_copy(ones_hbm, ones_v)
    pltpu.sync_copy(zcol_hbm, z_v)
    pltpu.sync_copy(z_v, acc_sh.at[pl.ds(sid * D1_LO, D1_LO)])
    plsc.subcore_barrier()

    for m in range(3):
        idx_start(m, m)

    def phase(k, m):
        idx_wait(m)
        pltpu.sync_copy(ones_v, acc_sh.at[idx_v.at[m]], add=True)
        idx_start(jnp.minimum(k + 3, T_MAIN - 1), m)

    def body(j, _):
        k = 3 * j
        for m in range(3):
            phase(k + m, m)
        return 0

    lax.fori_loop(0, T_MAIN // 3, body, 0)
    for m in range(3):
        idx_wait(m)

    @pl.when(wid < NTAIL)
    def _():
        base = (NFULL + wid) * CHUNK
        pltpu.sync_copy(dst_hbm.at[pl.ds(base, CHUNK)], idx_v.at[0])
        pltpu.sync_copy(ones_v, acc_sh.at[idx_v.at[0]], add=True)

    plsc.subcore_barrier()

    pltpu.sync_copy(acc_sh.at[pl.ds(sid * D1_LO, D1_LO)],
                    deg_out.at[cid].at[pl.ds(sid * D1_LO, D1_LO)])


@functools.partial(
    pl.kernel,
    mesh=_MESH,
    out_type=jax.ShapeDtypeStruct((2, N, 128), jnp.float32),
    scratch_types=[
        pltpu.VMEM((3, CHUNK), jnp.int32),
        pltpu.VMEM((3, CHUNK), jnp.int32),
        pltpu.VMEM((CHUNK, 128), jnp.float32),
        pltpu.VMEM((CHUNK, 128), jnp.float32),
        pltpu.VMEM((CHUNK, 128), jnp.float32),
        pltpu.VMEM_SHARED((N, 128), jnp.float32),
        pltpu.SemaphoreType.DMA,
        pltpu.SemaphoreType.DMA,
        pltpu.SemaphoreType.DMA,
        pltpu.SemaphoreType.DMA,
        pltpu.SemaphoreType.DMA,
        pltpu.SemaphoreType.DMA,
        pltpu.SemaphoreType.DMA,
        pltpu.SemaphoreType.DMA,
        pltpu.SemaphoreType.DMA,
        pltpu.SemaphoreType.DMA,
        pltpu.SemaphoreType.DMA,
        pltpu.SemaphoreType.DMA,
    ],
)
def _sc_prop(g_hbm, src_hbm, dst_hbm, zrows_hbm, acc_out,
             src_v, dst_v, rows_0, rows_1, rows_2, acc_sh,
             sem_s0, sem_s1, sem_s2, sem_d0, sem_d1, sem_d2,
             sem_g0, sem_g1, sem_g2, sem_c0, sem_c1, sem_c2):
    cid = lax.axis_index("c")
    sid = lax.axis_index("s")
    wid = sid * 2 + cid

    rows = (rows_0, rows_1, rows_2)
    sem_s = (sem_s0, sem_s1, sem_s2)
    sem_d = (sem_d0, sem_d1, sem_d2)
    sem_g = (sem_g0, sem_g1, sem_g2)
    sem_c = (sem_c0, sem_c1, sem_c2)

    def src_start(k, m):
        base = (wid + k * NW) * CHUNK
        pltpu.async_copy(src_hbm.at[pl.ds(base, CHUNK)], src_v.at[m], sem_s[m])

    def src_wait(m):
        pltpu.make_async_copy(src_hbm.at[pl.ds(0, CHUNK)], src_v.at[m],
                              sem_s[m]).wait()

    def dst_start(k, m):
        base = (wid + k * NW) * CHUNK
        pltpu.async_copy(dst_hbm.at[pl.ds(base, CHUNK)], dst_v.at[m], sem_d[m])

    def dst_wait(m):
        pltpu.make_async_copy(dst_hbm.at[pl.ds(0, CHUNK)], dst_v.at[m],
                              sem_d[m]).wait()

    def gather_start(m):
        pltpu.async_copy(g_hbm.at[src_v.at[m]], rows[m], sem_g[m])

    def gather_wait(m):
        pltpu.make_async_copy(g_hbm.at[src_v.at[m]], rows[m], sem_g[m]).wait()

    def scat_start(m):
        pltpu.async_copy(rows[m], acc_sh.at[dst_v.at[m]], sem_c[m], add=True)

    def scat_wait(m):
        pltpu.make_async_copy(rows[m], acc_sh.at[dst_v.at[m]], sem_c[m]).wait()

    pltpu.sync_copy(zrows_hbm, rows_0)

    @pl.when(sid < 15)
    def _():
        row0 = sid * ROWS_LO
        for off, size in ((0, 128), (128, 128), (256, 128), (384, 128),
                          (512, 112)):
            pltpu.sync_copy(rows_0.at[pl.ds(0, size)],
                            acc_sh.at[pl.ds(row0 + off, size)])

    @pl.when(sid == 15)
    def _():
        for off in (0, 128, 256, 384, 512):
            pltpu.sync_copy(rows_0, acc_sh.at[pl.ds(ROW_SPLIT + off, 128)])

    plsc.subcore_barrier()

    def phase(k, m, first=False):
        m2 = (m + 2) % 3
        dst_wait(m)
        gather_wait(m)
        scat_start(m)
        src_start(jnp.minimum(k + 3, T_MAIN - 1), m)
        if not first:
            scat_wait(m2)
        dst_start(jnp.minimum(k + 2, T_MAIN - 1), m2)
        src_wait(m2)
        gather_start(m2)

    src_start(0, 0)
    src_start(1, 1)
    src_start(2, 2)
    dst_start(0, 0)
    dst_start(1, 1)
    src_wait(0)
    gather_start(0)
    src_wait(1)
    gather_start(1)

    phase(0, 0, first=True)

    def body(j, _):
        k = 3 * j + 1
        phase(k, 1)
        phase(k + 1, 2)
        phase(k + 2, 0)
        return 0

    lax.fori_loop(0, (T_MAIN - 3) // 3, body, 0)
    phase(T_MAIN - 2, 1)
    phase(T_MAIN - 1, 2)
    scat_wait(2)
    gather_wait(0)
    gather_wait(1)
    src_wait(2)
    dst_wait(0)
    dst_wait(1)

    @pl.when(wid < NTAIL)
    def _():
        base = (NFULL + wid) * CHUNK
        pltpu.sync_copy(src_hbm.at[pl.ds(base, CHUNK)], src_v.at[0])
        pltpu.sync_copy(dst_hbm.at[pl.ds(base, CHUNK)], dst_v.at[0])
        pltpu.async_copy(g_hbm.at[src_v.at[0]], rows_0, sem_g0).wait()
        pltpu.sync_copy(rows_0, acc_sh.at[dst_v.at[0]], add=True)

    plsc.subcore_barrier()

    @pl.when(sid < 15)
    def _():
        pltpu.sync_copy(acc_sh.at[pl.ds(sid * ROWS_LO, ROWS_LO)],
                        acc_out.at[cid].at[pl.ds(sid * ROWS_LO, ROWS_LO)])

    @pl.when(sid == 15)
    def _():
        pltpu.sync_copy(acc_sh.at[pl.ds(ROW_SPLIT, ROWS_HI)],
                        acc_out.at[cid].at[pl.ds(ROW_SPLIT, ROWS_HI)])


_BLK = 2000


def _row_spec(width):
    return pl.BlockSpec((_BLK, width), lambda i: (i, 0))


def _full_spec(shape):
    nd = len(shape)
    return pl.BlockSpec(shape, lambda i: (0,) * nd)


def _tc_call(body, out_shapes, in_specs, out_specs):
    return pl.pallas_call(
        body,
        grid=(N // _BLK,),
        in_specs=in_specs,
        out_specs=out_specs,
        out_shape=out_shapes,
    )


def _k0_body(x_ref, w_ref, h_ref):
    h_ref[...] = jnp.dot(x_ref[...], w_ref[...],
                         preferred_element_type=jnp.float32)


def _k1_body(degp_ref, h_ref, dinv_ref, g_ref):
    deg = degp_ref[0] + degp_ref[1] + 1.0
    dinv = lax.rsqrt(deg)
    dinv_ref[...] = dinv
    g_ref[...] = dinv * h_ref[...]


def _k2_body(ap_ref, g_ref, dinv_ref, b1_ref, wml_ref, g2_ref):
    dinv = dinv_ref[...]
    h = jnp.tanh(dinv * (ap_ref[0] + ap_ref[1] + g_ref[...]) + b1_ref[...])
    g2_ref[...] = dinv * jnp.dot(h, wml_ref[...],
                                 preferred_element_type=jnp.float32)


def _k3_body(ap_ref, g_ref, dinv_ref, bm_ref, bl_ref, noise_ref, wd1_ref,
             mean_ref, logvar_ref, z_ref, g3_ref):
    dinv = dinv_ref[...]
    t = dinv * (ap_ref[0] + ap_ref[1] + g_ref[...])
    mean = t[:, :64] + bm_ref[...]
    logvar = t[:, 64:] + bl_ref[...]
    z = noise_ref[...] * jnp.exp(0.5 * logvar) + mean
    mean_ref[...] = mean
    logvar_ref[...] = logvar
    z_ref[...] = z
    g3_ref[...] = dinv * jnp.dot(z, wd1_ref[...],
                                 preferred_element_type=jnp.float32)


def _k4_body(ap_ref, g_ref, dinv_ref, bd1_ref, wd2_ref, g4_ref):
    dinv = dinv_ref[...]
    hd = jnp.tanh(dinv * (ap_ref[0] + ap_ref[1] + g_ref[...]) + bd1_ref[...])
    g4_ref[...] = dinv * jnp.dot(hd, wd2_ref[...],
                                 preferred_element_type=jnp.float32)


def _k5_body(ap_ref, g_ref, dinv_ref, bd2_ref, out_ref):
    out_ref[...] = (dinv_ref[...] * (ap_ref[0] + ap_ref[1] + g_ref[...])
                    + bd2_ref[...])


def _pair_spec(width):
    return pl.BlockSpec((2, _BLK, width), lambda i: (0, i, 0))


def kernel(feature, edge_index, W1, b1, Wm, bm, Wl, bl, Wd1, bd1, Wd2, bd2):
    f32 = jnp.float32
    src = edge_index[0]
    dst = edge_index[1]

    ones_col = jnp.ones((CHUNK,), f32)
    zcol = jnp.zeros((D1_LO,), f32)
    zrows = jnp.zeros((CHUNK, 128), f32)
    noise = jax.random.normal(jax.random.key(42), (N, 64), dtype=f32)
    Wml = jnp.concatenate([Wm, Wl], axis=1)

    h1 = _tc_call(
        _k0_body,
        jax.ShapeDtypeStruct((N, 128), f32),
        [_row_spec(128), _full_spec((128, 128))],
        _row_spec(128),
    )(feature, W1)

    deg_parts = _sc_degree(dst, ones_col, zcol)[:, :N, None]

    dinv, g1 = _tc_call(
        _k1_body,
        [jax.ShapeDtypeStruct((N, 1), f32), jax.ShapeDtypeStruct((N, 128), f32)],
        [_pair_spec(1), _row_spec(128)],
        [_row_spec(1), _row_spec(128)],
    )(deg_parts, h1)

    acc1 = _sc_prop(g1, src, dst, zrows)

    g2 = _tc_call(
        _k2_body,
        jax.ShapeDtypeStruct((N, 128), f32),
        [_pair_spec(128), _row_spec(128), _row_spec(1), _full_spec((128,)),
         _full_spec((128, 128))],
        _row_spec(128),
    )(acc1, g1, dinv, b1, Wml)

    acc2 = _sc_prop(g2, src, dst, zrows)

    mean, logvar, z, g3 = _tc_call(
        _k3_body,
        [jax.ShapeDtypeStruct((N, 64), f32), jax.ShapeDtypeStruct((N, 64), f32),
         jax.ShapeDtypeStruct((N, 64), f32), jax.ShapeDtypeStruct((N, 128), f32)],
        [_pair_spec(128), _row_spec(128), _row_spec(1), _full_spec((64,)),
         _full_spec((64,)), _row_spec(64), _full_spec((64, 128))],
        [_row_spec(64), _row_spec(64), _row_spec(64), _row_spec(128)],
    )(acc2, g2, dinv, bm, bl, noise, Wd1)

    acc3 = _sc_prop(g3, src, dst, zrows)

    g4 = _tc_call(
        _k4_body,
        jax.ShapeDtypeStruct((N, 128), f32),
        [_pair_spec(128), _row_spec(128), _row_spec(1), _full_spec((128,)),
         _full_spec((128, 128))],
        _row_spec(128),
    )(acc3, g3, dinv, bd1, Wd2)

    acc4 = _sc_prop(g4, src, dst, zrows)

    out = _tc_call(
        _k5_body,
        jax.ShapeDtypeStruct((N, 128), f32),
        [_pair_spec(128), _row_spec(128), _row_spec(1), _full_spec((128,))],
        _row_spec(128),
    )(acc4, g4, dinv, bd2)

    return (z, mean, logvar, out)


# speedup vs baseline: 33.2520x; 1.0542x over previous
import functools

import jax
import jax.numpy as jnp
from jax import lax
from jax.experimental import pallas as pl
from jax.experimental.pallas import tpu as pltpu
from jax.experimental.pallas import tpu_sc as plsc

N = 10000
E = 320000
CHUNK = 128
NCHUNKS = E // CHUNK
NW = 32
ROWS_LO = 624
ROWS_HI = 640
ROW_SPLIT = 15 * ROWS_LO
D1_LO = 640
NPAD1 = 16 * D1_LO

_MESH = plsc.VectorSubcoreMesh(core_axis_name="c", subcore_axis_name="s")

NFULL = (NCHUNKS // NW) * NW
NTAIL = NCHUNKS - NFULL
T_MAIN = NFULL // NW


@functools.partial(
    pl.kernel,
    mesh=_MESH,
    out_type=jax.ShapeDtypeStruct((2, NPAD1), jnp.float32),
    scratch_types=[
        pltpu.VMEM((3, CHUNK), jnp.int32),
        pltpu.VMEM((CHUNK,), jnp.float32),
        pltpu.VMEM((D1_LO,), jnp.float32),
        pltpu.VMEM_SHARED((NPAD1,), jnp.float32),
        pltpu.SemaphoreType.DMA,
        pltpu.SemaphoreType.DMA,
        pltpu.SemaphoreType.DMA,
    ],
)
def _sc_degree(dst_hbm, ones_hbm, zcol_hbm, deg_out, idx_v, ones_v, z_v,
               acc_sh, sem_0, sem_1, sem_2):
    cid = lax.axis_index("c")
    sid = lax.axis_index("s")
    wid = sid * 2 + cid
    sems = (sem_0, sem_1, sem_2)

    def idx_start(k, m):
        base = (wid + k * NW) * CHUNK
        pltpu.async_copy(dst_hbm.at[pl.ds(base, CHUNK)], idx_v.at[m], sems[m])

    def idx_wait(m):
        pltpu.make_async_copy(dst_hbm.at[pl.ds(0, CHUNK)], idx_v.at[m],
                              sems[m]).wait()

    pltpu.sync_copy(ones_hbm, ones_v)
    pltpu.sync_copy(zcol_hbm, z_v)
    pltpu.sync_copy(z_v, acc_sh.at[pl.ds(sid * D1_LO, D1_LO)])
    plsc.subcore_barrier()

    for m in range(3):
        idx_start(m, m)

    def phase(k, m):
        idx_wait(m)
        pltpu.sync_copy(ones_v, acc_sh.at[idx_v.at[m]], add=True)
        idx_start(jnp.minimum(k + 3, T_MAIN - 1), m)

    def body(j, _):
        k = 3 * j
        for m in range(3):
            phase(k + m, m)
        return 0

    lax.fori_loop(0, T_MAIN // 3, body, 0)
    for m in range(3):
        idx_wait(m)

    @pl.when(wid < NTAIL)
    def _():
        base = (NFULL + wid) * CHUNK
        pltpu.sync_copy(dst_hbm.at[pl.ds(base, CHUNK)], idx_v.at[0])
        pltpu.sync_copy(ones_v, acc_sh.at[idx_v.at[0]], add=True)

    plsc.subcore_barrier()

    pltpu.sync_copy(acc_sh.at[pl.ds(sid * D1_LO, D1_LO)],
                    deg_out.at[cid].at[pl.ds(sid * D1_LO, D1_LO)])


def _make_sc_prop(FW):
  @functools.partial(
    pl.kernel,
    mesh=_MESH,
    compiler_params=pltpu.CompilerParams(use_tc_tiling_on_sc=(FW == 128)),
    out_type=jax.ShapeDtypeStruct((2, N, FW), jnp.float32),
    scratch_types=[
        pltpu.VMEM((3, CHUNK), jnp.int32),
        pltpu.VMEM((3, CHUNK), jnp.int32),
        pltpu.VMEM((CHUNK, FW), jnp.float32),
        pltpu.VMEM((CHUNK, FW), jnp.float32),
        pltpu.VMEM((CHUNK, FW), jnp.float32),
        pltpu.VMEM_SHARED((N, FW), jnp.float32),
        pltpu.SemaphoreType.DMA,
        pltpu.SemaphoreType.DMA,
        pltpu.SemaphoreType.DMA,
        pltpu.SemaphoreType.DMA,
        pltpu.SemaphoreType.DMA,
        pltpu.SemaphoreType.DMA,
        pltpu.SemaphoreType.DMA,
        pltpu.SemaphoreType.DMA,
        pltpu.SemaphoreType.DMA,
        pltpu.SemaphoreType.DMA,
        pltpu.SemaphoreType.DMA,
        pltpu.SemaphoreType.DMA,
    ],
  )
  def _sc_prop(g_hbm, src_hbm, dst_hbm, zrows_hbm, acc_out,
               src_v, dst_v, rows_0, rows_1, rows_2, acc_sh,
               sem_s0, sem_s1, sem_s2, sem_d0, sem_d1, sem_d2,
               sem_g0, sem_g1, sem_g2, sem_c0, sem_c1, sem_c2):
    cid = lax.axis_index("c")
    sid = lax.axis_index("s")
    wid = sid * 2 + cid

    rows = (rows_0, rows_1, rows_2)
    sem_s = (sem_s0, sem_s1, sem_s2)
    sem_d = (sem_d0, sem_d1, sem_d2)
    sem_g = (sem_g0, sem_g1, sem_g2)
    sem_c = (sem_c0, sem_c1, sem_c2)

    def src_start(k, m):
        base = (wid + k * NW) * CHUNK
        pltpu.async_copy(src_hbm.at[pl.ds(base, CHUNK)], src_v.at[m], sem_s[m])

    def src_wait(m):
        pltpu.make_async_copy(src_hbm.at[pl.ds(0, CHUNK)], src_v.at[m],
                              sem_s[m]).wait()

    def dst_start(k, m):
        base = (wid + k * NW) * CHUNK
        pltpu.async_copy(dst_hbm.at[pl.ds(base, CHUNK)], dst_v.at[m], sem_d[m])

    def dst_wait(m):
        pltpu.make_async_copy(dst_hbm.at[pl.ds(0, CHUNK)], dst_v.at[m],
                              sem_d[m]).wait()

    def gather_start(m):
        pltpu.async_copy(g_hbm.at[src_v.at[m]], rows[m], sem_g[m])

    def gather_wait(m):
        pltpu.make_async_copy(g_hbm.at[src_v.at[m]], rows[m], sem_g[m]).wait()

    def scat_start(m):
        pltpu.async_copy(rows[m], acc_sh.at[dst_v.at[m]], sem_c[m], add=True)

    def scat_wait(m):
        pltpu.make_async_copy(rows[m], acc_sh.at[dst_v.at[m]], sem_c[m]).wait()

    pltpu.sync_copy(zrows_hbm, rows_0)

    @pl.when(sid < 15)
    def _():
        row0 = sid * ROWS_LO
        for off, size in ((0, 128), (128, 128), (256, 128), (384, 128),
                          (512, 112)):
            pltpu.sync_copy(rows_0.at[pl.ds(0, size)],
                            acc_sh.at[pl.ds(row0 + off, size)])

    @pl.when(sid == 15)
    def _():
        for off in (0, 128, 256, 384, 512):
            pltpu.sync_copy(rows_0, acc_sh.at[pl.ds(ROW_SPLIT + off, 128)])

    plsc.subcore_barrier()

    def phase(k, m, first=False):
        m2 = (m + 2) % 3
        dst_wait(m)
        gather_wait(m)
        scat_start(m)
        src_start(jnp.minimum(k + 3, T_MAIN - 1), m)
        if not first:
            scat_wait(m2)
        dst_start(jnp.minimum(k + 2, T_MAIN - 1), m2)
        src_wait(m2)
        gather_start(m2)

    src_start(0, 0)
    src_start(1, 1)
    src_start(2, 2)
    dst_start(0, 0)
    dst_start(1, 1)
    src_wait(0)
    gather_start(0)
    src_wait(1)
    gather_start(1)

    phase(0, 0, first=True)

    def body(j, _):
        k = 3 * j + 1
        phase(k, 1)
        phase(k + 1, 2)
        phase(k + 2, 0)
        return 0

    lax.fori_loop(0, (T_MAIN - 3) // 3, body, 0)
    phase(T_MAIN - 2, 1)
    phase(T_MAIN - 1, 2)
    scat_wait(2)
    gather_wait(0)
    gather_wait(1)
    src_wait(2)
    dst_wait(0)
    dst_wait(1)

    @pl.when(wid < NTAIL)
    def _():
        base = (NFULL + wid) * CHUNK
        pltpu.sync_copy(src_hbm.at[pl.ds(base, CHUNK)], src_v.at[0])
        pltpu.sync_copy(dst_hbm.at[pl.ds(base, CHUNK)], dst_v.at[0])
        pltpu.async_copy(g_hbm.at[src_v.at[0]], rows_0, sem_g0).wait()
        pltpu.sync_copy(rows_0, acc_sh.at[dst_v.at[0]], add=True)

    plsc.subcore_barrier()

    @pl.when(sid < 15)
    def _():
        pltpu.sync_copy(acc_sh.at[pl.ds(sid * ROWS_LO, ROWS_LO)],
                        acc_out.at[cid].at[pl.ds(sid * ROWS_LO, ROWS_LO)])

    @pl.when(sid == 15)
    def _():
        pltpu.sync_copy(acc_sh.at[pl.ds(ROW_SPLIT, ROWS_HI)],
                        acc_out.at[cid].at[pl.ds(ROW_SPLIT, ROWS_HI)])

  return _sc_prop


_sc_prop = _make_sc_prop(128)
_sc_prop64 = _make_sc_prop(64)


_BLK = 2000


def _row_spec(width):
    return pl.BlockSpec((_BLK, width), lambda i: (i, 0))


def _full_spec(shape):
    nd = len(shape)
    return pl.BlockSpec(shape, lambda i: (0,) * nd)


def _tc_call(body, out_shapes, in_specs, out_specs):
    return pl.pallas_call(
        body,
        grid=(N // _BLK,),
        in_specs=in_specs,
        out_specs=out_specs,
        out_shape=out_shapes,
    )


def _k0_body(x_ref, w_ref, h_ref):
    h_ref[...] = jnp.dot(x_ref[...], w_ref[...],
                         preferred_element_type=jnp.float32)


def _k1_body(degp_ref, h_ref, dinv_ref, g_ref):
    deg = degp_ref[0] + degp_ref[1] + 1.0
    dinv = lax.rsqrt(deg)
    dinv_ref[...] = dinv
    g_ref[...] = dinv * h_ref[...]


def _k2_body(ap_ref, g_ref, dinv_ref, b1_ref, wml_ref, g2_ref):
    dinv = dinv_ref[...]
    h = jnp.tanh(dinv * (ap_ref[0] + ap_ref[1] + g_ref[...]) + b1_ref[...])
    g2_ref[...] = dinv * jnp.dot(h, wml_ref[...],
                                 preferred_element_type=jnp.float32)


def _k3_body(ap_ref, g_ref, dinv_ref, bm_ref, bl_ref, noise_ref,
             mean_ref, logvar_ref, z_ref, gz_ref):
    dinv = dinv_ref[...]
    t = dinv * (ap_ref[0] + ap_ref[1] + g_ref[...])
    mean = t[:, :64] + bm_ref[...]
    logvar = t[:, 64:] + bl_ref[...]
    z = noise_ref[...] * jnp.exp(0.5 * logvar) + mean
    mean_ref[...] = mean
    logvar_ref[...] = logvar
    z_ref[...] = z
    gz_ref[...] = dinv * z


def _k4_body(ap_ref, g_ref, dinv_ref, wd1_ref, bd1_ref, wd2_ref, g4_ref):
    dinv = dinv_ref[...]
    pz = dinv * (ap_ref[0] + ap_ref[1] + g_ref[...])
    hd = jnp.tanh(jnp.dot(pz, wd1_ref[...],
                          preferred_element_type=jnp.float32) + bd1_ref[...])
    g4_ref[...] = dinv * jnp.dot(hd, wd2_ref[...],
                                 preferred_element_type=jnp.float32)


def _k5_body(ap_ref, g_ref, dinv_ref, bd2_ref, out_ref):
    out_ref[...] = (dinv_ref[...] * (ap_ref[0] + ap_ref[1] + g_ref[...])
                    + bd2_ref[...])


def _pair_spec(width):
    return pl.BlockSpec((2, _BLK, width), lambda i: (0, i, 0))


def kernel(feature, edge_index, W1, b1, Wm, bm, Wl, bl, Wd1, bd1, Wd2, bd2):
    f32 = jnp.float32
    src = edge_index[0]
    dst = edge_index[1]

    ones_col = jnp.ones((CHUNK,), f32)
    zcol = jnp.zeros((D1_LO,), f32)
    zrows = jnp.zeros((CHUNK, 128), f32)
    zrows64 = jnp.zeros((CHUNK, 64), f32)
    noise = jax.random.normal(jax.random.key(42), (N, 64), dtype=f32)
    Wml = jnp.concatenate([Wm, Wl], axis=1)

    h1 = _tc_call(
        _k0_body,
        jax.ShapeDtypeStruct((N, 128), f32),
        [_row_spec(128), _full_spec((128, 128))],
        _row_spec(128),
    )(feature, W1)

    deg_parts = _sc_degree(dst, ones_col, zcol)[:, :N, None]

    dinv, g1 = _tc_call(
        _k1_body,
        [jax.ShapeDtypeStruct((N, 1), f32), jax.ShapeDtypeStruct((N, 128), f32)],
        [_pair_spec(1), _row_spec(128)],
        [_row_spec(1), _row_spec(128)],
    )(deg_parts, h1)

    acc1 = _sc_prop(g1, src, dst, zrows)

    g2 = _tc_call(
        _k2_body,
        jax.ShapeDtypeStruct((N, 128), f32),
        [_pair_spec(128), _row_spec(128), _row_spec(1), _full_spec((128,)),
         _full_spec((128, 128))],
        _row_spec(128),
    )(acc1, g1, dinv, b1, Wml)

    acc2 = _sc_prop(g2, src, dst, zrows)

    mean, logvar, z, gz = _tc_call(
        _k3_body,
        [jax.ShapeDtypeStruct((N, 64), f32), jax.ShapeDtypeStruct((N, 64), f32),
         jax.ShapeDtypeStruct((N, 64), f32), jax.ShapeDtypeStruct((N, 64), f32)],
        [_pair_spec(128), _row_spec(128), _row_spec(1), _full_spec((64,)),
         _full_spec((64,)), _row_spec(64)],
        [_row_spec(64), _row_spec(64), _row_spec(64), _row_spec(64)],
    )(acc2, g2, dinv, bm, bl, noise)

    acc3 = _sc_prop64(gz, src, dst, zrows64)

    g4 = _tc_call(
        _k4_body,
        jax.ShapeDtypeStruct((N, 128), f32),
        [_pair_spec(64), _row_spec(64), _row_spec(1), _full_spec((64, 128)),
         _full_spec((128,)), _full_spec((128, 128))],
        _row_spec(128),
    )(acc3, gz, dinv, Wd1, bd1, Wd2)

    acc4 = _sc_prop(g4, src, dst, zrows)

    out = _tc_call(
        _k5_body,
        jax.ShapeDtypeStruct((N, 128), f32),
        [_pair_spec(128), _row_spec(128), _row_spec(1), _full_spec((128,))],
        _row_spec(128),
    )(acc4, g4, dinv, bd2)

    return (z, mean, logvar, out)
